# baseline (speedup 1.0000x reference)
import functools

import jax
import jax.numpy as jnp
from jax import lax
from jax.experimental import pallas as pl
from jax.experimental.pallas import tpu as pltpu
from jax.experimental.pallas import tpu_sc as plsc

B, L, D = 1024, 512, 64
NC, NS = 2, 16
NW = NC * NS
ROWS_PER_W = B // NW
NIDX = 4
CHUNK = L // NIDX
LANES = 16


def _body(x_hbm, tok_hbm, pos_hbm, out_hbm, idx_v, buf_v, pos_v, sem):
    c = lax.axis_index("c")
    s = lax.axis_index("s")
    wid = s * NC + c
    base = wid * ROWS_PER_W

    pltpu.sync_copy(pos_hbm, pos_v)

    def do_row(r, _):
        row = base + r
        pltpu.sync_copy(x_hbm.at[row], idx_v)
        copies = [
            pltpu.async_copy(
                tok_hbm.at[idx_v.at[j]], buf_v.at[pl.ds(j * CHUNK, CHUNK)], sem
            )
            for j in range(NIDX)
        ]
        for cp in copies:
            cp.wait()

        def add_row(i, _):
            for k in range(D // LANES):
                sl = pl.ds(k * LANES, LANES)
                buf_v[i, sl] = buf_v[i, sl] + pos_v[i, sl]
            return 0

        lax.fori_loop(0, L, add_row, 0)
        pltpu.sync_copy(buf_v, out_hbm.at[row])
        return 0

    lax.fori_loop(0, ROWS_PER_W, do_row, 0)


_emb = functools.partial(
    pl.kernel,
    out_type=jax.ShapeDtypeStruct((B, L, D), jnp.float32),
    mesh=plsc.VectorSubcoreMesh(core_axis_name="c", subcore_axis_name="s"),
    scratch_types=[
        pltpu.VMEM((NIDX, CHUNK), jnp.int32),
        pltpu.VMEM((L, D), jnp.float32),
        pltpu.VMEM((L, D), jnp.float32),
        pltpu.SemaphoreType.DMA,
    ],
    compiler_params=pltpu.CompilerParams(use_tc_tiling_on_sc=False),
)(_body)


@jax.jit
def kernel(x, token_emb, pos_emb):
    x2 = x.astype(jnp.int32).reshape(B, NIDX, CHUNK)
    return _emb(x2, token_emb, pos_emb)


# speedup vs baseline: 1.2137x; 1.2137x over previous
import functools

import jax
import jax.numpy as jnp
from jax import lax
from jax.experimental import pallas as pl
from jax.experimental.pallas import tpu as pltpu
from jax.experimental.pallas import tpu_sc as plsc

B, L, D = 1024, 512, 64
DP = 128
NC, NS = 2, 16
NW = NC * NS
ROWS_PER_W = B // NW
NQ = 4
QL = L // NQ
LANES = 16


def _body(x_hbm, tok_hbm, pos_hbm, out_hbm, idx_v, bufa, bufb, pos_v, sem_g, sem_w):
    c = lax.axis_index("c")
    s = lax.axis_index("s")
    wid = s * NC + c
    base = wid * ROWS_PER_W

    pltpu.sync_copy(pos_hbm, pos_v)

    def add_quarter(q, buf):
        def add_tok(t, _):
            for k in range(D // LANES):
                sl = pl.ds(k * LANES, LANES)
                buf[t, sl] = buf[t, sl] + pos_v[q * QL + t, sl]
            return 0

        lax.fori_loop(0, QL, add_tok, 0)

    def do_row(r, _):
        row = base + r
        pltpu.sync_copy(x_hbm.at[row], idx_v)
        bufs = [bufa, bufb, bufa, bufb]

        def gather(q):
            return pltpu.async_copy(
                tok_hbm.at[idx_v.at[pl.ds(q * QL, QL)]], bufs[q], sem_g
            )

        def store(q):
            return pltpu.async_copy(
                bufs[q], out_hbm.at[row, pl.ds(q * QL, QL)], sem_w
            )

        g0 = gather(0)
        g0.wait()
        g1 = gather(1)
        add_quarter(0, bufa)
        w0 = store(0)
        g1.wait()
        w0.wait()
        g2 = gather(2)
        add_quarter(1, bufb)
        w1 = store(1)
        g2.wait()
        w1.wait()
        g3 = gather(3)
        add_quarter(2, bufa)
        w2 = store(2)
        g3.wait()
        add_quarter(3, bufb)
        w3 = store(3)
        w2.wait()
        w3.wait()
        return 0

    lax.fori_loop(0, ROWS_PER_W, do_row, 0)


_emb = functools.partial(
    pl.kernel,
    out_type=jax.ShapeDtypeStruct((B, L, DP), jnp.float32),
    mesh=plsc.VectorSubcoreMesh(core_axis_name="c", subcore_axis_name="s"),
    scratch_types=[
        pltpu.VMEM((L,), jnp.int32),
        pltpu.VMEM((QL, DP), jnp.float32),
        pltpu.VMEM((QL, DP), jnp.float32),
        pltpu.VMEM((L, DP), jnp.float32),
        pltpu.SemaphoreType.DMA,
        pltpu.SemaphoreType.DMA,
    ],
    compiler_params=pltpu.CompilerParams(use_tc_tiling_on_sc=True),
)(_body)


@jax.jit
def kernel(x, token_emb, pos_emb):
    tok_pad = jnp.pad(token_emb, ((0, 0), (0, DP - D)))
    pos_pad = jnp.pad(pos_emb, ((0, 0), (0, DP - D)))
    out = _emb(x.astype(jnp.int32), tok_pad, pos_pad)
    return out[:, :, :D]
